# baseline (speedup 1.0000x reference)
import jax
import jax.numpy as jnp
from jax.experimental import pallas as pl
from jax.experimental.pallas import tpu as pltpu

_N = 5000
_B = 512
_NP = 5120
_K = _NP // _B
_NMS_THRESH = 0.5


def _nms_body(rx1, ry1, rx2, ry2, rarea,
              cx1, cy1, cx2, cy2, carea,
              data,
              out,
              sup):
    k = pl.program_id(0)

    @pl.when(k == 0)
    def _():
        sup[...] = jnp.zeros_like(sup)

    c0 = k * _B

    def s_block(a, masked):
        x1r = rx1[pl.ds(a, _B), :]
        y1r = ry1[pl.ds(a, _B), :]
        x2r = rx2[pl.ds(a, _B), :]
        y2r = ry2[pl.ds(a, _B), :]
        ar = rarea[pl.ds(a, _B), :]
        x1c = cx1[:, pl.ds(c0, _B)]
        y1c = cy1[:, pl.ds(c0, _B)]
        x2c = cx2[:, pl.ds(c0, _B)]
        y2c = cy2[:, pl.ds(c0, _B)]
        ac = carea[:, pl.ds(c0, _B)]
        wx = jnp.clip(jnp.minimum(x2r, x2c) - jnp.maximum(x1r, x1c), 0.0)
        wy = jnp.clip(jnp.minimum(y2r, y2c) - jnp.maximum(y1r, y1c), 0.0)
        inter = wx * wy
        union = ar + ac - inter
        iou = inter / jnp.maximum(union, 1e-9)
        hit = iou > _NMS_THRESH
        if masked:
            jidx = a + jax.lax.broadcasted_iota(jnp.int32, (_B, _B), 0)
            iidx = c0 + jax.lax.broadcasted_iota(jnp.int32, (_B, _B), 1)
            hit = hit & (jidx > iidx)
        return jnp.where(hit, 1.0, 0.0)

    s_kk = s_block(c0, masked=True)
    alive = sup[pl.ds(c0, _B), :] < 0.5
    keep0 = jnp.where(alive, 1.0, 0.0)

    def jcond(c):
        return c[1]

    def jbody(c):
        keep, _ = c
        cnt = jax.lax.dot_general(
            s_kk, keep, (((1,), (0,)), ((), ())),
            preferred_element_type=jnp.float32)
        keep_n = jnp.where(alive & (cnt < 0.5), 1.0, 0.0)
        return keep_n, jnp.any(keep_n != keep)

    keep, _ = jax.lax.while_loop(jcond, jbody, (keep0, jnp.array(True)))

    def pbody(m, _):
        a = m * _B
        cnt = jax.lax.dot_general(
            s_block(a, masked=False), keep, (((1,), (0,)), ((), ())),
            preferred_element_type=jnp.float32)
        sup[pl.ds(a, _B), :] += cnt
        return 0

    jax.lax.fori_loop(k + 1, _K, pbody, 0)

    out[pl.ds(c0, _B), :] = data[pl.ds(c0, _B), :] * keep


def kernel(boxes, scores, classes):
    scores = scores.astype(jnp.float32)
    max_coord = jnp.max(boxes) + 1.0
    sorted_neg, x1, y1, x2, y2, cls = jax.lax.sort(
        (-scores, boxes[:, 0], boxes[:, 1], boxes[:, 2], boxes[:, 3],
         classes.astype(jnp.float32)),
        num_keys=1, is_stable=True)
    s = -sorted_neg

    pad = _NP - _N
    ob = jnp.stack([x1, y1, x2, y2], axis=1)
    obp = jnp.pad(ob, ((0, pad), (0, 0)))
    off = cls * max_coord
    offp = jnp.pad(off, (0, pad))
    bp = obp + offp[:, None]
    sp = jnp.pad(s, (0, pad))
    area = (bp[:, 2] - bp[:, 0]) * (bp[:, 3] - bp[:, 1])

    rowdat = jnp.concatenate([bp, area[:, None]], axis=1)
    coldat = rowdat.T
    rows = [rowdat[:, i:i + 1] for i in range(5)]
    cols = [coldat[i:i + 1, :] for i in range(5)]
    data = jnp.concatenate(
        [obp, sp[:, None], jnp.zeros((_NP, 3), jnp.float32)], axis=1)

    full_rc = pl.BlockSpec((_NP, 1), lambda k: (0, 0))
    full_cc = pl.BlockSpec((1, _NP), lambda k: (0, 0))
    full_d = pl.BlockSpec((_NP, 8), lambda k: (0, 0))

    out = pl.pallas_call(
        _nms_body,
        grid=(_K,),
        in_specs=[full_rc] * 5 + [full_cc] * 5 + [full_d],
        out_specs=full_d,
        out_shape=jax.ShapeDtypeStruct((_NP, 8), jnp.float32),
        scratch_shapes=[pltpu.VMEM((_NP, 1), jnp.float32)],
        compiler_params=pltpu.CompilerParams(
            dimension_semantics=("arbitrary",)),
    )(*rows, *cols, data)

    return out[:_N, :5]


# speedup vs baseline: 1.0537x; 1.0537x over previous
import jax
import jax.numpy as jnp
from jax.experimental import pallas as pl
from jax.experimental.pallas import tpu as pltpu

_N = 5000
_B = 512
_NP = 5120
_K = _NP // _B
_NMS_THRESH = 0.5


def _nms_body(rx1, ry1, rx2, ry2, rarea,
              cx1, cy1, cx2, cy2, carea,
              data,
              out,
              sup):
    k = pl.program_id(0)

    @pl.when(k == 0)
    def _():
        sup[...] = jnp.zeros_like(sup)

    c0 = k * _B
    x1c = cx1[:, pl.ds(c0, _B)]
    y1c = cy1[:, pl.ds(c0, _B)]
    x2c = cx2[:, pl.ds(c0, _B)]
    y2c = cy2[:, pl.ds(c0, _B)]
    ac = carea[:, pl.ds(c0, _B)]

    def iou_rows(a, x2cm):
        x1r = rx1[pl.ds(a, _B), :]
        y1r = ry1[pl.ds(a, _B), :]
        x2r = rx2[pl.ds(a, _B), :]
        y2r = ry2[pl.ds(a, _B), :]
        ar = rarea[pl.ds(a, _B), :]
        wx = jnp.clip(jnp.minimum(x2r, x2cm) - jnp.maximum(x1r, x1c), 0.0)
        wy = jnp.clip(jnp.minimum(y2r, y2c) - jnp.maximum(y1r, y1c), 0.0)
        inter = wx * wy
        union = ar + ac - inter
        return inter / jnp.maximum(union, 1e-9)

    jidx = jax.lax.broadcasted_iota(jnp.int32, (_B, _B), 0)
    iidx = jax.lax.broadcasted_iota(jnp.int32, (_B, _B), 1)
    s_kk = jnp.where((iou_rows(c0, x2c) > _NMS_THRESH) & (jidx > iidx),
                     1.0, 0.0)
    alive = sup[pl.ds(c0, _B), :] <= _NMS_THRESH
    keep0 = jnp.where(alive, 1.0, 0.0)

    def jcond(c):
        return c[1]

    def jbody(c):
        keep, _ = c
        cnt = jax.lax.dot_general(
            s_kk, keep, (((1,), (0,)), ((), ())),
            preferred_element_type=jnp.float32)
        keep_n = jnp.where(alive & (cnt < 0.5), 1.0, 0.0)
        return keep_n, jnp.any(keep_n != keep)

    keep, _ = jax.lax.while_loop(jcond, jbody, (keep0, jnp.array(True)))

    keep_row = jnp.transpose(keep)
    x2m = jnp.where(keep_row > 0.5, x2c, -1e9)

    def pbody(m, _):
        a = m * _B
        mx = jnp.max(iou_rows(a, x2m), axis=1, keepdims=True)
        sup[pl.ds(a, _B), :] = jnp.maximum(sup[pl.ds(a, _B), :], mx)
        return 0

    jax.lax.fori_loop(k + 1, _K, pbody, 0)

    out[pl.ds(c0, _B), :] = data[pl.ds(c0, _B), :] * keep


def kernel(boxes, scores, classes):
    scores = scores.astype(jnp.float32)
    max_coord = jnp.max(boxes) + 1.0
    sorted_neg, x1, y1, x2, y2, cls = jax.lax.sort(
        (-scores, boxes[:, 0], boxes[:, 1], boxes[:, 2], boxes[:, 3],
         classes.astype(jnp.float32)),
        num_keys=1, is_stable=True)
    s = -sorted_neg

    pad = _NP - _N
    ob = jnp.stack([x1, y1, x2, y2], axis=1)
    obp = jnp.pad(ob, ((0, pad), (0, 0)))
    off = cls * max_coord
    offp = jnp.pad(off, (0, pad))
    bp = obp + offp[:, None]
    sp = jnp.pad(s, (0, pad))
    area = (bp[:, 2] - bp[:, 0]) * (bp[:, 3] - bp[:, 1])

    rowdat = jnp.concatenate([bp, area[:, None]], axis=1)
    coldat = rowdat.T
    rows = [rowdat[:, i:i + 1] for i in range(5)]
    cols = [coldat[i:i + 1, :] for i in range(5)]
    data = jnp.concatenate(
        [obp, sp[:, None], jnp.zeros((_NP, 3), jnp.float32)], axis=1)

    full_rc = pl.BlockSpec((_NP, 1), lambda k: (0, 0))
    full_cc = pl.BlockSpec((1, _NP), lambda k: (0, 0))
    full_d = pl.BlockSpec((_NP, 8), lambda k: (0, 0))

    out = pl.pallas_call(
        _nms_body,
        grid=(_K,),
        in_specs=[full_rc] * 5 + [full_cc] * 5 + [full_d],
        out_specs=full_d,
        out_shape=jax.ShapeDtypeStruct((_NP, 8), jnp.float32),
        scratch_shapes=[pltpu.VMEM((_NP, 1), jnp.float32)],
        compiler_params=pltpu.CompilerParams(
            dimension_semantics=("arbitrary",)),
    )(*rows, *cols, data)

    return out[:_N, :5]


# speedup vs baseline: 1.7034x; 1.6165x over previous
import jax
import jax.numpy as jnp
from jax.experimental import pallas as pl
from jax.experimental.pallas import tpu as pltpu

_N = 5000
_B = 512
_NP = 5120
_K = _NP // _B
_NMS_THRESH = 0.5


def _nms_body(packed,
              data8,
              out,
              sup):
    k = pl.program_id(0)

    @pl.when(k == 0)
    def _():
        sup[...] = jnp.zeros_like(sup)

    c0 = k * _B
    bt = jnp.transpose(packed[:, pl.ds(c0, _B)])
    x1r = bt[:, 0:1]
    y1r = bt[:, 1:2]
    x2r = bt[:, 2:3]
    y2r = bt[:, 3:4]
    ar = bt[:, 4:5]

    def iou_cols(a, x2rm):
        x1c = packed[0:1, pl.ds(a, _B)]
        y1c = packed[1:2, pl.ds(a, _B)]
        x2c = packed[2:3, pl.ds(a, _B)]
        y2c = packed[3:4, pl.ds(a, _B)]
        ac = packed[4:5, pl.ds(a, _B)]
        wx = jnp.clip(jnp.minimum(x2rm, x2c) - jnp.maximum(x1r, x1c), 0.0)
        wy = jnp.clip(jnp.minimum(y2r, y2c) - jnp.maximum(y1r, y1c), 0.0)
        inter = wx * wy
        union = ar + ac - inter
        return inter / jnp.maximum(union, 1e-9)

    siota = jax.lax.broadcasted_iota(jnp.int32, (_B, _B), 0)
    liota = jax.lax.broadcasted_iota(jnp.int32, (_B, _B), 1)
    s_kk = jnp.where((iou_cols(c0, x2r) > _NMS_THRESH) & (liota > siota),
                     1.0, 0.0)
    alive = sup[:, pl.ds(c0, _B)] <= _NMS_THRESH
    keep0 = jnp.where(alive, 1.0, 0.0)

    def jcond(c):
        return c[1]

    def jbody(c):
        keep, _ = c
        cnt = jax.lax.dot_general(
            keep, s_kk, (((1,), (0,)), ((), ())),
            preferred_element_type=jnp.float32)
        keep_n = jnp.where(alive & (cnt < 0.5), 1.0, 0.0)
        return keep_n, jnp.any(keep_n != keep)

    keep, _ = jax.lax.while_loop(jcond, jbody, (keep0, jnp.array(True)))

    keep_col = jnp.transpose(keep)
    x2rm = jnp.where(keep_col > 0.5, x2r, -1e9)

    def pbody(m, _):
        a = m * _B
        mx = jnp.max(iou_cols(a, x2rm), axis=0, keepdims=True)
        sup[:, pl.ds(a, _B)] = jnp.maximum(sup[:, pl.ds(a, _B)], mx)
        return 0

    jax.lax.fori_loop(k + 1, _K, pbody, 0)

    out[:, pl.ds(c0, _B)] = data8[:, pl.ds(c0, _B)] * keep


def kernel(boxes, scores, classes):
    scores = scores.astype(jnp.float32)
    max_coord = jnp.max(boxes) + 1.0
    sorted_neg, x1, y1, x2, y2, cls = jax.lax.sort(
        (-scores, boxes[:, 0], boxes[:, 1], boxes[:, 2], boxes[:, 3],
         classes.astype(jnp.float32)),
        num_keys=1, is_stable=True)
    s = -sorted_neg

    off = cls * max_coord
    x1o, y1o, x2o, y2o = x1 + off, y1 + off, x2 + off, y2 + off
    area = (x2o - x1o) * (y2o - y1o)

    padspec = ((0, 3), (0, _NP - _N))
    packed = jnp.pad(jnp.stack([x1o, y1o, x2o, y2o, area]), padspec)
    data8 = jnp.pad(jnp.stack([x1, y1, x2, y2, s]), padspec)

    full8 = pl.BlockSpec((8, _NP), lambda k: (0, 0))
    out8 = pl.pallas_call(
        _nms_body,
        grid=(_K,),
        in_specs=[full8, full8],
        out_specs=full8,
        out_shape=jax.ShapeDtypeStruct((8, _NP), jnp.float32),
        scratch_shapes=[pltpu.VMEM((1, _NP), jnp.float32)],
        compiler_params=pltpu.CompilerParams(
            dimension_semantics=("arbitrary",)),
    )(packed, data8)

    return jnp.transpose(out8[:5, :_N])


# speedup vs baseline: 1.8545x; 1.0887x over previous
import jax
import jax.numpy as jnp
from jax.experimental import pallas as pl
from jax.experimental.pallas import tpu as pltpu

_N = 5000
_B = 1024
_NP = 5120
_K = _NP // _B
_NMS_THRESH = 0.5


def _nms_body(packed,
              data8,
              out,
              sup):
    k = pl.program_id(0)

    @pl.when(k == 0)
    def _():
        sup[...] = jnp.zeros_like(sup)

    c0 = k * _B
    bt = jnp.transpose(packed[:, pl.ds(c0, _B)])
    x1r = bt[:, 0:1]
    y1r = bt[:, 1:2]
    x2r = bt[:, 2:3]
    y2r = bt[:, 3:4]
    ar = bt[:, 4:5]

    def iou_cols(a, x2rm):
        x1c = packed[0:1, pl.ds(a, _B)]
        y1c = packed[1:2, pl.ds(a, _B)]
        x2c = packed[2:3, pl.ds(a, _B)]
        y2c = packed[3:4, pl.ds(a, _B)]
        ac = packed[4:5, pl.ds(a, _B)]
        wx = jnp.clip(jnp.minimum(x2rm, x2c) - jnp.maximum(x1r, x1c), 0.0)
        wy = jnp.clip(jnp.minimum(y2r, y2c) - jnp.maximum(y1r, y1c), 0.0)
        inter = wx * wy
        union = ar + ac - inter
        return inter / jnp.maximum(union, 1e-9)

    siota = jax.lax.broadcasted_iota(jnp.int32, (_B, _B), 0)
    liota = jax.lax.broadcasted_iota(jnp.int32, (_B, _B), 1)
    s_kk = jnp.where((iou_cols(c0, x2r) > _NMS_THRESH) & (liota > siota),
                     1.0, 0.0)
    alive = sup[:, pl.ds(c0, _B)] <= _NMS_THRESH
    keep0 = jnp.where(alive, 1.0, 0.0)

    def jcond(c):
        return c[1]

    def jbody(c):
        keep, _ = c
        cnt = jax.lax.dot_general(
            keep, s_kk, (((1,), (0,)), ((), ())),
            preferred_element_type=jnp.float32)
        keep_n = jnp.where(alive & (cnt < 0.5), 1.0, 0.0)
        return keep_n, jnp.any(keep_n != keep)

    keep, _ = jax.lax.while_loop(jcond, jbody, (keep0, jnp.array(True)))

    keep_col = jnp.transpose(keep)
    x2rm = jnp.where(keep_col > 0.5, x2r, -1e9)

    def pbody(m, _):
        a = m * _B
        mx = jnp.max(iou_cols(a, x2rm), axis=0, keepdims=True)
        sup[:, pl.ds(a, _B)] = jnp.maximum(sup[:, pl.ds(a, _B)], mx)
        return 0

    jax.lax.fori_loop(k + 1, _K, pbody, 0)

    out[:, pl.ds(c0, _B)] = data8[:, pl.ds(c0, _B)] * keep


def kernel(boxes, scores, classes):
    scores = scores.astype(jnp.float32)
    max_coord = jnp.max(boxes) + 1.0
    sorted_neg, x1, y1, x2, y2, cls = jax.lax.sort(
        (-scores, boxes[:, 0], boxes[:, 1], boxes[:, 2], boxes[:, 3],
         classes.astype(jnp.float32)),
        num_keys=1, is_stable=True)
    s = -sorted_neg

    off = cls * max_coord
    x1o, y1o, x2o, y2o = x1 + off, y1 + off, x2 + off, y2 + off
    area = (x2o - x1o) * (y2o - y1o)

    padspec = ((0, 3), (0, _NP - _N))
    packed = jnp.pad(jnp.stack([x1o, y1o, x2o, y2o, area]), padspec)
    data8 = jnp.pad(jnp.stack([x1, y1, x2, y2, s]), padspec)

    full8 = pl.BlockSpec((8, _NP), lambda k: (0, 0))
    out8 = pl.pallas_call(
        _nms_body,
        grid=(_K,),
        in_specs=[full8, full8],
        out_specs=full8,
        out_shape=jax.ShapeDtypeStruct((8, _NP), jnp.float32),
        scratch_shapes=[pltpu.VMEM((1, _NP), jnp.float32)],
        compiler_params=pltpu.CompilerParams(
            dimension_semantics=("arbitrary",)),
    )(packed, data8)

    return jnp.transpose(out8[:5, :_N])
